# baseline (speedup 1.0000x reference)
import functools

import jax
import jax.numpy as jnp
from jax import lax
from jax.experimental import pallas as pl
from jax.experimental.pallas import tpu as pltpu
from jax.experimental.pallas import tpu_sc as plsc

SAMPLE_B = 1024
WRITE_B = 4096
D = 3 * 32 * 32

NC, NS = 2, 16
NW = NC * NS
ROWS_PER = SAMPLE_B // NW
CHUNK = 16
NCHUNK = ROWS_PER // CHUNK
PAD = 8


def _winner_body(idx_ref, s_ref, w_ref):
    ix = idx_ref[...]
    s = s_ref[...].reshape(1, 128)
    eq = ix == s
    j = lax.broadcasted_iota(jnp.int32, (WRITE_B, 128), 0)
    cand = jnp.where(eq, j, -1)
    w_ref[...] = jnp.max(cand, axis=0, keepdims=True).reshape(1, 1, 128)


def _winner_tc(idx, sample_idx):
    idx2 = idx.reshape(WRITE_B, 1)
    s3 = sample_idx.reshape(SAMPLE_B // 128, 1, 128)
    grid = SAMPLE_B // 128
    w = pl.pallas_call(
        _winner_body,
        grid=(grid,),
        in_specs=[
            pl.BlockSpec((WRITE_B, 1), lambda i: (0, 0)),
            pl.BlockSpec((1, 1, 128), lambda i: (i, 0, 0)),
        ],
        out_specs=pl.BlockSpec((1, 1, 128), lambda i: (i, 0, 0)),
        out_shape=jax.ShapeDtypeStruct((SAMPLE_B // 128, 1, 128), jnp.int32),
    )(idx2, s3)
    return w.reshape(SAMPLE_B)


def _merge_body(sidx_sref, win_sref, mem_ref, val_ref, out_ref, sem):
    def issue(i, carry):
        w = win_sref[i]
        s = sidx_sref[i]

        @pl.when(w >= 0)
        def _():
            pltpu.make_async_copy(
                val_ref.at[jnp.maximum(w, 0)], out_ref.at[i], sem).start()

        @pl.when(w < 0)
        def _():
            pltpu.make_async_copy(mem_ref.at[s], out_ref.at[i], sem).start()

        return carry

    lax.fori_loop(0, SAMPLE_B, issue, 0)

    def drain(i, carry):
        pltpu.make_async_copy(mem_ref.at[0], out_ref.at[0], sem).wait()
        return carry

    lax.fori_loop(0, SAMPLE_B, drain, 0)


def _merge_tc(mem, val, sample_idx, winner):
    row = mem.shape[1:]

    grid_spec = pltpu.PrefetchScalarGridSpec(
        num_scalar_prefetch=2,
        grid=(1,),
        in_specs=[
            pl.BlockSpec(memory_space=pltpu.HBM),
            pl.BlockSpec(memory_space=pltpu.HBM),
        ],
        out_specs=pl.BlockSpec(memory_space=pltpu.HBM),
        scratch_shapes=[pltpu.SemaphoreType.DMA],
    )
    return pl.pallas_call(
        _merge_body,
        grid_spec=grid_spec,
        out_shape=jax.ShapeDtypeStruct((SAMPLE_B,) + row, jnp.float32),
    )(sample_idx, winner, mem, val)


def _sc_gather(mem2, val2, sample_idx, winner):
    mesh = plsc.VectorSubcoreMesh(core_axis_name="c", subcore_axis_name="s")

    @functools.partial(
        pl.kernel,
        mesh=mesh,
        out_type=jax.ShapeDtypeStruct((SAMPLE_B, D), jnp.float32),
        scratch_types=[
            pltpu.VMEM((ROWS_PER,), jnp.int32),
            pltpu.VMEM((ROWS_PER,), jnp.int32),
            pltpu.VMEM((CHUNK, D), jnp.float32),
            pltpu.VMEM((CHUNK, D), jnp.float32),
            pltpu.SemaphoreType.DMA,
            pltpu.SemaphoreType.DMA,
            pltpu.SemaphoreType.DMA,
            pltpu.SemaphoreType.DMA,
        ],
    )
    def k(mem_hbm, val_hbm, sidx_hbm, win_hbm, out_hbm,
          sidx_v, win_v, buf0, buf1, semg0, semg1, semw0, semw1):
        wid = lax.axis_index("s") * NC + lax.axis_index("c")
        base = wid * ROWS_PER

        pltpu.sync_copy(sidx_hbm.at[pl.ds(base, ROWS_PER)], sidx_v)
        pltpu.sync_copy(win_hbm.at[pl.ds(base, ROWS_PER)], win_v)

        bufs = (buf0, buf1)
        semg = (semg0, semg1)
        semw = (semw0, semw1)

        gathers = []
        for c in range(NCHUNK):
            sidx = sidx_v[pl.ds(c * CHUNK, CHUNK)]
            gathers.append(
                pltpu.async_copy(mem_hbm.at[sidx], bufs[c % 2], semg[c % 2]))

        writes = []
        for c in range(NCHUNK):
            buf = bufs[c % 2]
            gathers[c].wait()
            wv = win_v[pl.ds(c * CHUNK, CHUNK)]
            for rr in range(CHUNK):
                w_r = wv[rr]

                @pl.when(w_r >= 0)
                def _(w_r=w_r, buf=buf, rr=rr):
                    pltpu.sync_copy(val_hbm.at[w_r], buf.at[rr])

            writes.append(pltpu.async_copy(
                buf, out_hbm.at[pl.ds(base + c * CHUNK, CHUNK)], semw[c % 2]))
        for wr in writes:
            wr.wait()

    return k(mem2, val2, sample_idx, winner)


RCHUNK = 4


def _sc_native_merge(mem, val, sample_idx, winner):
    mesh = plsc.VectorSubcoreMesh(core_axis_name="c", subcore_axis_name="s")
    row_shape = mem.shape[1:]

    @functools.partial(
        pl.kernel,
        mesh=mesh,
        out_type=jax.ShapeDtypeStruct((SAMPLE_B,) + row_shape, jnp.float32),
        scratch_types=[
            pltpu.VMEM((ROWS_PER,), jnp.int32),
            pltpu.VMEM((ROWS_PER,), jnp.int32),
            pltpu.VMEM((RCHUNK,) + row_shape, jnp.float32),
            pltpu.VMEM((RCHUNK,) + row_shape, jnp.float32),
            pltpu.SemaphoreType.DMA,
            pltpu.SemaphoreType.DMA,
        ],
    )
    def k(mem_hbm, val_hbm, sidx_hbm, win_hbm, out_hbm,
          sidx_v, win_v, buf0, buf1, sem0, sem1):
        wid = lax.axis_index("s") * NC + lax.axis_index("c")
        base = wid * ROWS_PER
        pltpu.sync_copy(sidx_hbm.at[pl.ds(base, ROWS_PER)], sidx_v)
        pltpu.sync_copy(win_hbm.at[pl.ds(base, ROWS_PER)], win_v)

        bufs = (buf0, buf1)
        sems = (sem0, sem1)
        nchunk = ROWS_PER // RCHUNK
        pending = [None, None]

        for c in range(nchunk):
            buf, sem = bufs[c % 2], sems[c % 2]
            if pending[c % 2] is not None:
                pending[c % 2].wait()
            wv = win_v[pl.ds(c * RCHUNK // 16 * 16, 16)]
            sv = sidx_v[pl.ds(c * RCHUNK // 16 * 16, 16)]
            for rr in range(RCHUNK):
                r = c * RCHUNK + rr
                w_r = wv[r % 16]
                s_r = sv[r % 16]

                @pl.when(w_r >= 0)
                def _(w_r=w_r, buf=buf, rr=rr):
                    pltpu.sync_copy(val_hbm.at[w_r], buf.at[rr])

                @pl.when(w_r < 0)
                def _(s_r=s_r, buf=buf, rr=rr):
                    pltpu.sync_copy(mem_hbm.at[s_r], buf.at[rr])

            pending[c % 2] = pltpu.async_copy(
                buf, out_hbm.at[pl.ds(base + c * RCHUNK, RCHUNK)], sem)

        pending[nchunk % 2].wait()
        pending[(nchunk + 1) % 2].wait()

    return k(mem, val, sample_idx, winner)


RING = 8
LAG = 4


def _ring_body(sidx_sref, win_sref, mem_ref, val_ref, out_ref, buf, gsem, wsem):
    def it(i, carry):
        slot = lax.rem(i, RING)

        @pl.when(i >= RING)
        def _():
            pltpu.make_async_copy(
                buf.at[slot], out_ref.at[i - RING], wsem.at[slot]).wait()

        w = win_sref[i]
        s = sidx_sref[i]

        @pl.when(w >= 0)
        def _():
            pltpu.make_async_copy(
                val_ref.at[jnp.maximum(w, 0)], buf.at[slot],
                gsem.at[slot]).start()

        @pl.when(w < 0)
        def _():
            pltpu.make_async_copy(
                mem_ref.at[s], buf.at[slot], gsem.at[slot]).start()

        j = i - LAG

        @pl.when(j >= 0)
        def _():
            slotj = lax.rem(j, RING)
            pltpu.make_async_copy(
                mem_ref.at[0], buf.at[slotj], gsem.at[slotj]).wait()
            pltpu.make_async_copy(
                buf.at[slotj], out_ref.at[j], wsem.at[slotj]).start()

        return carry

    lax.fori_loop(0, SAMPLE_B, it, 0)

    for j in range(SAMPLE_B - LAG, SAMPLE_B):
        slotj = j % RING
        pltpu.make_async_copy(
            mem_ref.at[0], buf.at[slotj], gsem.at[slotj]).wait()
        pltpu.make_async_copy(
            buf.at[slotj], out_ref.at[j], wsem.at[slotj]).start()
    for j in range(SAMPLE_B - RING, SAMPLE_B):
        slotj = j % RING
        pltpu.make_async_copy(
            buf.at[slotj], out_ref.at[j], wsem.at[slotj]).wait()


def _merge_ring_tc(mem, val, sample_idx, winner):
    row = mem.shape[1:]
    grid_spec = pltpu.PrefetchScalarGridSpec(
        num_scalar_prefetch=2,
        grid=(1,),
        in_specs=[
            pl.BlockSpec(memory_space=pltpu.HBM),
            pl.BlockSpec(memory_space=pltpu.HBM),
        ],
        out_specs=pl.BlockSpec(memory_space=pltpu.HBM),
        scratch_shapes=[
            pltpu.VMEM((RING,) + row, jnp.float32),
            pltpu.SemaphoreType.DMA((RING,)),
            pltpu.SemaphoreType.DMA((RING,)),
        ],
    )
    return pl.pallas_call(
        _ring_body,
        grid_spec=grid_spec,
        out_shape=jax.ShapeDtypeStruct((SAMPLE_B,) + row, jnp.float32),
    )(sample_idx, winner, mem, val)


def kernel(mem, idx, val, sample_idx):
    winner = _winner_tc(idx, sample_idx)
    return _merge_ring_tc(mem, val, sample_idx, winner)


# speedup vs baseline: 3.6747x; 3.6747x over previous
import functools

import jax
import jax.numpy as jnp
from jax import lax
from jax.experimental import pallas as pl
from jax.experimental.pallas import tpu as pltpu
from jax.experimental.pallas import tpu_sc as plsc

SAMPLE_B = 1024
WRITE_B = 4096
D = 3 * 32 * 32

NC, NS = 2, 16
NW = NC * NS
ROWS_PER = SAMPLE_B // NW
CHUNK = 16
NCHUNK = ROWS_PER // CHUNK
PAD = 8


def _winner_body(idx_ref, s_ref, w_ref):
    ix = idx_ref[...]
    s = s_ref[...].reshape(1, 128)
    eq = ix == s
    j = lax.broadcasted_iota(jnp.int32, (WRITE_B, 128), 0)
    cand = jnp.where(eq, j, -1)
    w_ref[...] = jnp.max(cand, axis=0, keepdims=True).reshape(1, 1, 128)


def _winner_tc(idx, sample_idx):
    idx2 = idx.reshape(WRITE_B, 1)
    s3 = sample_idx.reshape(SAMPLE_B // 128, 1, 128)
    grid = SAMPLE_B // 128
    w = pl.pallas_call(
        _winner_body,
        grid=(grid,),
        in_specs=[
            pl.BlockSpec((WRITE_B, 1), lambda i: (0, 0)),
            pl.BlockSpec((1, 1, 128), lambda i: (i, 0, 0)),
        ],
        out_specs=pl.BlockSpec((1, 1, 128), lambda i: (i, 0, 0)),
        out_shape=jax.ShapeDtypeStruct((SAMPLE_B // 128, 1, 128), jnp.int32),
    )(idx2, s3)
    return w.reshape(SAMPLE_B)


def _merge_body(sidx_sref, win_sref, mem_ref, val_ref, out_ref, sem):
    def issue(i, carry):
        w = win_sref[i]
        s = sidx_sref[i]

        @pl.when(w >= 0)
        def _():
            pltpu.make_async_copy(
                val_ref.at[jnp.maximum(w, 0)], out_ref.at[i], sem).start()

        @pl.when(w < 0)
        def _():
            pltpu.make_async_copy(mem_ref.at[s], out_ref.at[i], sem).start()

        return carry

    lax.fori_loop(0, SAMPLE_B, issue, 0)

    def drain(i, carry):
        pltpu.make_async_copy(mem_ref.at[0], out_ref.at[0], sem).wait()
        return carry

    lax.fori_loop(0, SAMPLE_B, drain, 0)


def _merge_tc(mem, val, sample_idx, winner):
    row = mem.shape[1:]

    grid_spec = pltpu.PrefetchScalarGridSpec(
        num_scalar_prefetch=2,
        grid=(1,),
        in_specs=[
            pl.BlockSpec(memory_space=pltpu.HBM),
            pl.BlockSpec(memory_space=pltpu.HBM),
        ],
        out_specs=pl.BlockSpec(memory_space=pltpu.HBM),
        scratch_shapes=[pltpu.SemaphoreType.DMA],
    )
    return pl.pallas_call(
        _merge_body,
        grid_spec=grid_spec,
        out_shape=jax.ShapeDtypeStruct((SAMPLE_B,) + row, jnp.float32),
    )(sample_idx, winner, mem, val)


def _sc_gather(mem2, val2, sample_idx, winner):
    mesh = plsc.VectorSubcoreMesh(core_axis_name="c", subcore_axis_name="s")

    @functools.partial(
        pl.kernel,
        mesh=mesh,
        out_type=jax.ShapeDtypeStruct((SAMPLE_B, D), jnp.float32),
        scratch_types=[
            pltpu.VMEM((ROWS_PER,), jnp.int32),
            pltpu.VMEM((ROWS_PER,), jnp.int32),
            pltpu.VMEM((CHUNK, D), jnp.float32),
            pltpu.VMEM((CHUNK, D), jnp.float32),
            pltpu.SemaphoreType.DMA,
            pltpu.SemaphoreType.DMA,
            pltpu.SemaphoreType.DMA,
            pltpu.SemaphoreType.DMA,
        ],
    )
    def k(mem_hbm, val_hbm, sidx_hbm, win_hbm, out_hbm,
          sidx_v, win_v, buf0, buf1, semg0, semg1, semw0, semw1):
        wid = lax.axis_index("s") * NC + lax.axis_index("c")
        base = wid * ROWS_PER

        pltpu.sync_copy(sidx_hbm.at[pl.ds(base, ROWS_PER)], sidx_v)
        pltpu.sync_copy(win_hbm.at[pl.ds(base, ROWS_PER)], win_v)

        bufs = (buf0, buf1)
        semg = (semg0, semg1)
        semw = (semw0, semw1)

        gathers = []
        for c in range(NCHUNK):
            sidx = sidx_v[pl.ds(c * CHUNK, CHUNK)]
            gathers.append(
                pltpu.async_copy(mem_hbm.at[sidx], bufs[c % 2], semg[c % 2]))

        writes = []
        for c in range(NCHUNK):
            buf = bufs[c % 2]
            gathers[c].wait()
            wv = win_v[pl.ds(c * CHUNK, CHUNK)]
            for rr in range(CHUNK):
                w_r = wv[rr]

                @pl.when(w_r >= 0)
                def _(w_r=w_r, buf=buf, rr=rr):
                    pltpu.sync_copy(val_hbm.at[w_r], buf.at[rr])

            writes.append(pltpu.async_copy(
                buf, out_hbm.at[pl.ds(base + c * CHUNK, CHUNK)], semw[c % 2]))
        for wr in writes:
            wr.wait()

    return k(mem2, val2, sample_idx, winner)


RCHUNK = 4


def _sc_native_merge(mem, val, sample_idx, winner):
    mesh = plsc.VectorSubcoreMesh(core_axis_name="c", subcore_axis_name="s")
    row_shape = mem.shape[1:]

    @functools.partial(
        pl.kernel,
        mesh=mesh,
        out_type=jax.ShapeDtypeStruct((SAMPLE_B,) + row_shape, jnp.float32),
        scratch_types=[
            pltpu.VMEM((ROWS_PER,), jnp.int32),
            pltpu.VMEM((ROWS_PER,), jnp.int32),
            pltpu.VMEM((RCHUNK,) + row_shape, jnp.float32),
            pltpu.VMEM((RCHUNK,) + row_shape, jnp.float32),
            pltpu.SemaphoreType.DMA,
            pltpu.SemaphoreType.DMA,
        ],
    )
    def k(mem_hbm, val_hbm, sidx_hbm, win_hbm, out_hbm,
          sidx_v, win_v, buf0, buf1, sem0, sem1):
        wid = lax.axis_index("s") * NC + lax.axis_index("c")
        base = wid * ROWS_PER
        pltpu.sync_copy(sidx_hbm.at[pl.ds(base, ROWS_PER)], sidx_v)
        pltpu.sync_copy(win_hbm.at[pl.ds(base, ROWS_PER)], win_v)

        bufs = (buf0, buf1)
        sems = (sem0, sem1)
        nchunk = ROWS_PER // RCHUNK
        pending = [None, None]

        for c in range(nchunk):
            buf, sem = bufs[c % 2], sems[c % 2]
            if pending[c % 2] is not None:
                pending[c % 2].wait()
            wv = win_v[pl.ds(c * RCHUNK // 16 * 16, 16)]
            sv = sidx_v[pl.ds(c * RCHUNK // 16 * 16, 16)]
            for rr in range(RCHUNK):
                r = c * RCHUNK + rr
                w_r = wv[r % 16]
                s_r = sv[r % 16]

                @pl.when(w_r >= 0)
                def _(w_r=w_r, buf=buf, rr=rr):
                    pltpu.sync_copy(val_hbm.at[w_r], buf.at[rr])

                @pl.when(w_r < 0)
                def _(s_r=s_r, buf=buf, rr=rr):
                    pltpu.sync_copy(mem_hbm.at[s_r], buf.at[rr])

            pending[c % 2] = pltpu.async_copy(
                buf, out_hbm.at[pl.ds(base + c * RCHUNK, RCHUNK)], sem)

        pending[nchunk % 2].wait()
        pending[(nchunk + 1) % 2].wait()

    return k(mem, val, sample_idx, winner)


def kernel(mem, idx, val, sample_idx):
    cap = mem.shape[0]
    mem2 = mem.reshape(cap, D)
    val2 = val.reshape(WRITE_B, D)
    winner = _winner_tc(idx, sample_idx)
    out2 = _sc_gather(mem2, val2, sample_idx, winner)
    return out2.reshape(SAMPLE_B, *mem.shape[1:])
